# baseline (speedup 1.0000x reference)
import functools

import jax
import jax.numpy as jnp
from jax import lax
from jax.experimental import pallas as pl
from jax.experimental.pallas import tpu as pltpu
from jax.experimental.pallas import tpu_sc as plsc

N = 10000
E = 320000
D = 128

NC, NS, L = 2, 16, 16
NW = NC * NS
CH = 128
NCHUNK = 80
EPW = NCHUNK * CH
E_PAD = NW * EPW
NP = 10240
TRASH = N
RPT = NP // NS
BN = 512
GRID = NP // BN

_f32 = jnp.float32


def _mesh():
    return plsc.VectorSubcoreMesh(core_axis_name="c", subcore_axis_name="s")


@functools.partial(
    pl.kernel,
    out_type=(
        jax.ShapeDtypeStruct((NC, 2, NP), _f32),
        jax.ShapeDtypeStruct((NW, NCHUNK, CH), jnp.int32),
        jax.ShapeDtypeStruct((NW, NCHUNK, CH), jnp.int32),
    ),
    mesh=_mesh(),
    compiler_params=pltpu.CompilerParams(use_tc_tiling_on_sc=False),
    scratch_types=[
        pltpu.VMEM((NCHUNK, CH), jnp.int32),
        pltpu.VMEM((NCHUNK, CH), jnp.int32),
        pltpu.VMEM((NCHUNK, CH), jnp.int32),
        pltpu.VMEM((NCHUNK, CH), jnp.int32),
        pltpu.VMEM((CH,), _f32),
        pltpu.VMEM((RPT,), _f32),
        pltpu.VMEM_SHARED((NP,), _f32),
        pltpu.VMEM_SHARED((NP,), _f32),
    ],
)
def _sc_hist(dst_hbm, rev_hbm, out_hbm, ist_hbm, its_hbm,
             dstb, revb, idxf, idxt, ones, zbuf, h_all, h_fwd):
    c = lax.axis_index("c")
    s = lax.axis_index("s")
    w = s * NC + c
    pltpu.sync_copy(dst_hbm.at[w], dstb)
    pltpu.sync_copy(rev_hbm.at[w], revb)
    for i in range(CH // L):
        ones[pl.ds(i * L, L)] = jnp.full((L,), 1.0, _f32)

    def zloop(i, t):
        zbuf[pl.ds(i * L, L)] = jnp.zeros((L,), _f32)
        return t

    lax.fori_loop(0, RPT // L, zloop, 0)

    trash = jnp.full((L,), TRASH, jnp.int32)

    def iloop(j, t):
        for i in range(CH // L):
            d = dstb[j, pl.ds(i * L, L)]
            r = revb[j, pl.ds(i * L, L)]
            idxf[j, pl.ds(i * L, L)] = jnp.where(r == 0, d, trash)
            idxt[j, pl.ds(i * L, L)] = jnp.where(r == 0, trash, d)
        return t

    lax.fori_loop(0, NCHUNK, iloop, 0)
    pltpu.sync_copy(idxf, ist_hbm.at[w])
    pltpu.sync_copy(idxt, its_hbm.at[w])

    pltpu.sync_copy(zbuf, h_all.at[pl.ds(s * RPT, RPT)])
    pltpu.sync_copy(zbuf, h_fwd.at[pl.ds(s * RPT, RPT)])
    plsc.subcore_barrier()

    def sloop(j, t):
        pltpu.sync_copy(ones, h_all.at[dstb.at[j]], add=True)
        pltpu.sync_copy(ones, h_fwd.at[idxf.at[j]], add=True)
        return t

    lax.fori_loop(0, NCHUNK, sloop, 0)
    plsc.subcore_barrier()
    pltpu.sync_copy(h_all.at[pl.ds(s * RPT, RPT)], out_hbm.at[c, 0, pl.ds(s * RPT, RPT)])
    pltpu.sync_copy(h_fwd.at[pl.ds(s * RPT, RPT)], out_hbm.at[c, 1, pl.ds(s * RPT, RPT)])


def _make_sc_layer(width):
    @functools.partial(
        pl.kernel,
        out_type=jax.ShapeDtypeStruct((NC, NP, width), _f32),
        mesh=_mesh(),
        compiler_params=pltpu.CompilerParams(use_tc_tiling_on_sc=False),
        scratch_types=[
            pltpu.VMEM((NCHUNK, CH), jnp.int32),
            pltpu.VMEM((NCHUNK, CH), jnp.int32),
            pltpu.VMEM((CH, width), _f32),
            pltpu.VMEM((32, width), _f32),
            pltpu.VMEM_SHARED((NP, width), _f32),
            pltpu.SemaphoreType.DMA,
        ],
    )
    def _sc_layer(src_hbm, dst_hbm, t_hbm, p_hbm,
                  srcb, dstb, rows0, zb, acc, gs0):
        c = lax.axis_index("c")
        s = lax.axis_index("s")
        w = s * NC + c
        pltpu.sync_copy(src_hbm.at[w], srcb)
        pltpu.sync_copy(dst_hbm.at[w], dstb)

        def zbloop(i, t):
            for q in range(width // L):
                zb[i, pl.ds(q * L, L)] = jnp.zeros((L,), _f32)
            return t

        lax.fori_loop(0, 32, zbloop, 0)

        def zsloop(i, t):
            pltpu.sync_copy(zb, acc.at[pl.ds(s * RPT + i * 32, 32)])
            return t

        lax.fori_loop(0, RPT // 32, zsloop, 0)
        plsc.subcore_barrier()

        def mloop(j, t):
            pltpu.async_copy(t_hbm.at[srcb.at[j]], rows0, gs0).wait()
            pltpu.sync_copy(rows0, acc.at[dstb.at[j]], add=True)
            return t

        lax.fori_loop(0, NCHUNK, mloop, 0)
        plsc.subcore_barrier()
        sl = pl.ds(s * RPT, RPT)
        pltpu.sync_copy(acc.at[sl], p_hbm.at[c, sl])

    return _sc_layer


_sc_gs64 = _make_sc_layer(64)
_sc_gs128 = _make_sc_layer(128)


def _prep1_body(hist_ref, x_ref, wcat_ref, tst_ref, tts_ref, ta_ref, dinv_ref):
    hp = hist_ref[...]
    deg_all = hp[0, 0] + hp[1, 0] + 1.0
    deg_fwd = hp[0, 1] + hp[1, 1] + 1.0
    deg_ts = deg_all - deg_fwd + 1.0
    d_st = lax.rsqrt(deg_fwd)
    d_ts = lax.rsqrt(deg_ts)
    d_a = lax.rsqrt(deg_all)
    h = jnp.dot(x_ref[...], wcat_ref[...], preferred_element_type=_f32)
    tst_ref[...] = d_st[:, None] * h[:, :64]
    tts_ref[...] = d_ts[:, None] * h[:, 64:128]
    ta_ref[...] = d_a[:, None] * h[:, 128:192]
    dinv_ref[...] = jnp.concatenate(
        [d_st[None], d_ts[None], d_a[None], jnp.zeros((5, d_st.shape[0]), _f32)], axis=0)


def _tc_prep1(hist, xp, wcat):
    return pl.pallas_call(
        _prep1_body,
        grid=(GRID,),
        in_specs=[
            pl.BlockSpec((2, 2, BN), lambda i: (0, 0, i)),
            pl.BlockSpec((BN, D), lambda i: (i, 0)),
            pl.BlockSpec((D, 192), lambda i: (0, 0)),
        ],
        out_specs=[
            pl.BlockSpec((BN, 64), lambda i: (i, 0)),
            pl.BlockSpec((BN, 64), lambda i: (i, 0)),
            pl.BlockSpec((BN, 64), lambda i: (i, 0)),
            pl.BlockSpec((8, BN), lambda i: (0, i)),
        ],
        out_shape=[
            jax.ShapeDtypeStruct((NP, 64), _f32),
            jax.ShapeDtypeStruct((NP, 64), _f32),
            jax.ShapeDtypeStruct((NP, 64), _f32),
            jax.ShapeDtypeStruct((8, NP), _f32),
        ],
    )(hist, xp, wcat)


def _mid1_body(pst, pts, pa, tst, tts, ta, dinv, bst, bts, ba, w2, t2o):
    dv = dinv[...]
    h_st = jnp.maximum(dv[0][:, None] * (pst[0] + pst[1] + tst[...]) + bst[...], 0.0)
    h_ts = jnp.maximum(dv[1][:, None] * (pts[0] + pts[1] + tts[...]) + bts[...], 0.0)
    h_a = jnp.maximum(dv[2][:, None] * (pa[0] + pa[1] + ta[...]) + ba[...], 0.0)
    h1 = jnp.concatenate([h_st, h_ts, h_a], axis=1)
    t2o[...] = dv[2][:, None] * jnp.dot(h1, w2[...], preferred_element_type=_f32)


def _tc_mid1(pst, pts, pa, tst, tts, ta, dinv, bst, bts, ba, w2):
    p = pl.BlockSpec((2, BN, 64), lambda i: (0, i, 0))
    t = pl.BlockSpec((BN, 64), lambda i: (i, 0))
    b = pl.BlockSpec((1, 64), lambda i: (0, 0))
    return pl.pallas_call(
        _mid1_body,
        grid=(GRID,),
        in_specs=[p, p, p, t, t, t,
                  pl.BlockSpec((8, BN), lambda i: (0, i)),
                  b, b, b,
                  pl.BlockSpec((192, 128), lambda i: (0, 0))],
        out_specs=pl.BlockSpec((BN, 128), lambda i: (i, 0)),
        out_shape=jax.ShapeDtypeStruct((NP, 128), _f32),
    )(pst, pts, pa, tst, tts, ta, dinv, bst, bts, ba, w2)


def _mid2_body(p2, t2, dinv, b2, w3, t3o):
    da = dinv[...][2]
    h2 = da[:, None] * (p2[0] + p2[1] + t2[...]) + b2[...]
    t3o[...] = da[:, None] * jnp.dot(h2, w3[...], preferred_element_type=_f32)


def _tc_mid2(p2, t2, dinv, b2, w3p):
    return pl.pallas_call(
        _mid2_body,
        grid=(GRID,),
        in_specs=[
            pl.BlockSpec((2, BN, 128), lambda i: (0, i, 0)),
            pl.BlockSpec((BN, 128), lambda i: (i, 0)),
            pl.BlockSpec((8, BN), lambda i: (0, i)),
            pl.BlockSpec((1, 128), lambda i: (0, 0)),
            pl.BlockSpec((128, 64), lambda i: (0, 0)),
        ],
        out_specs=pl.BlockSpec((BN, 64), lambda i: (i, 0)),
        out_shape=jax.ShapeDtypeStruct((NP, 64), _f32),
    )(p2, t2, dinv, b2, w3p)


def _fin_body(p3, t3, dinv, b3, out):
    da = dinv[...][2]
    h3 = da[:, None] * (p3[0] + p3[1] + t3[...]) + b3[...]
    col = lax.broadcasted_iota(jnp.int32, h3.shape, 1)
    valid = col < 40
    hm = jnp.where(valid, h3, jnp.full_like(h3, -jnp.inf))
    mx = jnp.max(hm, axis=1, keepdims=True)
    ex = jnp.where(valid, jnp.exp(h3 - mx), jnp.zeros_like(h3))
    lse = jnp.log(jnp.sum(ex, axis=1, keepdims=True)) + mx
    out[...] = h3 - lse


def _tc_fin(p3, t3, dinv, b3p):
    return pl.pallas_call(
        _fin_body,
        grid=(GRID,),
        in_specs=[
            pl.BlockSpec((2, BN, 64), lambda i: (0, i, 0)),
            pl.BlockSpec((BN, 64), lambda i: (i, 0)),
            pl.BlockSpec((8, BN), lambda i: (0, i)),
            pl.BlockSpec((1, 64), lambda i: (0, 0)),
        ],
        out_specs=pl.BlockSpec((BN, 64), lambda i: (i, 0)),
        out_shape=jax.ShapeDtypeStruct((NP, 64), _f32),
    )(p3, t3, dinv, b3p)


def kernel(x, edge_index, is_reversed, W_st1, b_st1, W_ts1, b_ts1, W1, b1,
           W2, b2, W3, b3):
    src = edge_index[0]
    dst = edge_index[1]
    rev = is_reversed.astype(jnp.int32)
    pad = E_PAD - E
    src_p = jnp.concatenate([src, jnp.zeros((pad,), jnp.int32)]).reshape(NW, NCHUNK, CH)
    dst_p = jnp.concatenate([dst, jnp.full((pad,), TRASH, jnp.int32)]).reshape(NW, NCHUNK, CH)
    rev_p = jnp.concatenate([rev, jnp.zeros((pad,), jnp.int32)]).reshape(NW, NCHUNK, CH)

    hist, ist_p, its_p = _sc_hist(dst_p, rev_p)
    xp = jnp.pad(x, ((0, NP - N), (0, 0)))
    wcat = jnp.concatenate([W_st1, W_ts1, W1], axis=1)
    tst, tts, ta, dinv = _tc_prep1(hist, xp, wcat)
    src_b, dst_b, ist_b, its_b = src_p, dst_p, ist_p, its_p
    pst = _sc_gs64(src_b, ist_b, tst)
    pts = _sc_gs64(src_b, its_b, tts)
    pa = _sc_gs64(src_b, dst_b, ta)
    t2 = _tc_mid1(pst, pts, pa, tst, tts, ta, dinv,
                  b_st1.reshape(1, 64), b_ts1.reshape(1, 64),
                  b1.reshape(1, 64), W2)
    p2 = _sc_gs128(src_b, dst_b, t2)
    w3p = jnp.pad(W3, ((0, 0), (0, 24)))
    t3 = _tc_mid2(p2, t2, dinv, b2.reshape(1, 128), w3p)
    p3 = _sc_gs64(src_b, dst_b, t3)
    b3p = jnp.pad(b3, (0, 24)).reshape(1, 64)
    out = _tc_fin(p3, t3, dinv, b3p)
    return out[:N, :40]


# speedup vs baseline: 1.4093x; 1.4093x over previous
import functools

import jax
import jax.numpy as jnp
from jax import lax
from jax.experimental import pallas as pl
from jax.experimental.pallas import tpu as pltpu
from jax.experimental.pallas import tpu_sc as plsc

N = 10000
E = 320000
D = 128

NC, NS, L = 2, 16, 16
NW = NC * NS
CH = 128
NCHUNK = 79
EPW = NCHUNK * CH
E_PAD = NW * EPW
NP = 10240
TRASH = N
RPT = NP // NS
BN = 512
GRID = NP // BN

_f32 = jnp.float32


def _mesh():
    return plsc.VectorSubcoreMesh(core_axis_name="c", subcore_axis_name="s")


@functools.partial(
    pl.kernel,
    out_type=(
        jax.ShapeDtypeStruct((NC, 2, NP), _f32),
        jax.ShapeDtypeStruct((NW, NCHUNK, CH), jnp.int32),
        jax.ShapeDtypeStruct((NW, NCHUNK, CH), jnp.int32),
    ),
    mesh=_mesh(),
    compiler_params=pltpu.CompilerParams(use_tc_tiling_on_sc=False),
    scratch_types=[
        pltpu.VMEM((NCHUNK, CH), jnp.int32),
        pltpu.VMEM((NCHUNK, CH), jnp.int32),
        pltpu.VMEM((NCHUNK, CH), jnp.int32),
        pltpu.VMEM((NCHUNK, CH), jnp.int32),
        pltpu.VMEM((CH,), _f32),
        pltpu.VMEM((RPT,), _f32),
        pltpu.VMEM_SHARED((NP,), _f32),
        pltpu.VMEM_SHARED((NP,), _f32),
    ],
)
def _sc_hist(dst_hbm, rev_hbm, out_hbm, ist_hbm, its_hbm,
             dstb, revb, idxf, idxt, ones, zbuf, h_all, h_fwd):
    c = lax.axis_index("c")
    s = lax.axis_index("s")
    w = s * NC + c
    pltpu.sync_copy(dst_hbm.at[w], dstb)
    pltpu.sync_copy(rev_hbm.at[w], revb)
    for i in range(CH // L):
        ones[pl.ds(i * L, L)] = jnp.full((L,), 1.0, _f32)

    def zloop(i, t):
        zbuf[pl.ds(i * L, L)] = jnp.zeros((L,), _f32)
        return t

    lax.fori_loop(0, RPT // L, zloop, 0)

    trash = jnp.full((L,), TRASH, jnp.int32)

    def iloop(j, t):
        for i in range(CH // L):
            d = dstb[j, pl.ds(i * L, L)]
            r = revb[j, pl.ds(i * L, L)]
            idxf[j, pl.ds(i * L, L)] = jnp.where(r == 0, d, trash)
            idxt[j, pl.ds(i * L, L)] = jnp.where(r == 0, trash, d)
        return t

    lax.fori_loop(0, NCHUNK, iloop, 0)
    pltpu.sync_copy(idxf, ist_hbm.at[w])
    pltpu.sync_copy(idxt, its_hbm.at[w])

    pltpu.sync_copy(zbuf, h_all.at[pl.ds(s * RPT, RPT)])
    pltpu.sync_copy(zbuf, h_fwd.at[pl.ds(s * RPT, RPT)])
    plsc.subcore_barrier()

    def sloop(j, t):
        pltpu.sync_copy(ones, h_all.at[dstb.at[j]], add=True)
        pltpu.sync_copy(ones, h_fwd.at[idxf.at[j]], add=True)
        return t

    lax.fori_loop(0, NCHUNK, sloop, 0)
    plsc.subcore_barrier()
    pltpu.sync_copy(h_all.at[pl.ds(s * RPT, RPT)], out_hbm.at[c, 0, pl.ds(s * RPT, RPT)])
    pltpu.sync_copy(h_fwd.at[pl.ds(s * RPT, RPT)], out_hbm.at[c, 1, pl.ds(s * RPT, RPT)])


def _make_sc_layer(width):
    @functools.partial(
        pl.kernel,
        out_type=jax.ShapeDtypeStruct((NC, NP, width), _f32),
        mesh=_mesh(),
        compiler_params=pltpu.CompilerParams(use_tc_tiling_on_sc=False),
        scratch_types=[
            pltpu.VMEM((NCHUNK, CH), jnp.int32),
            pltpu.VMEM((NCHUNK, CH), jnp.int32),
            pltpu.VMEM((CH, width), _f32),
            pltpu.VMEM((32, width), _f32),
            pltpu.VMEM_SHARED((NP, width), _f32),
            pltpu.SemaphoreType.DMA,
        ],
    )
    def _sc_layer(src_hbm, dst_hbm, t_hbm, p_hbm,
                  srcb, dstb, rows0, zb, acc, gs0):
        c = lax.axis_index("c")
        s = lax.axis_index("s")
        w = s * NC + c
        pltpu.sync_copy(src_hbm.at[w], srcb)
        pltpu.sync_copy(dst_hbm.at[w], dstb)

        def zbloop(i, t):
            for q in range(width // L):
                zb[i, pl.ds(q * L, L)] = jnp.zeros((L,), _f32)
            return t

        lax.fori_loop(0, 32, zbloop, 0)

        def zsloop(i, t):
            pltpu.sync_copy(zb, acc.at[pl.ds(s * RPT + i * 32, 32)])
            return t

        lax.fori_loop(0, RPT // 32, zsloop, 0)
        plsc.subcore_barrier()

        def mloop(j, t):
            pltpu.async_copy(t_hbm.at[srcb.at[j]], rows0, gs0).wait()
            pltpu.sync_copy(rows0, acc.at[dstb.at[j]], add=True)
            return t

        lax.fori_loop(0, NCHUNK, mloop, 0)
        plsc.subcore_barrier()
        sl = pl.ds(s * RPT, RPT)
        pltpu.sync_copy(acc.at[sl], p_hbm.at[c, sl])

    return _sc_layer


_sc_gs64 = _make_sc_layer(64)
_sc_gs128 = _make_sc_layer(128)


def _prep1_body(hist_ref, x_ref, wcat_ref, tst_ref, tts_ref, ta_ref, dinv_ref):
    hp = hist_ref[...]
    deg_all = hp[0, 0] + hp[1, 0] + 1.0
    deg_fwd = hp[0, 1] + hp[1, 1] + 1.0
    deg_ts = deg_all - deg_fwd + 1.0
    d_st = lax.rsqrt(deg_fwd)
    d_ts = lax.rsqrt(deg_ts)
    d_a = lax.rsqrt(deg_all)
    h = jnp.dot(x_ref[...], wcat_ref[...], preferred_element_type=_f32)
    tst_ref[...] = d_st[:, None] * h[:, :64]
    tts_ref[...] = d_ts[:, None] * h[:, 64:128]
    ta_ref[...] = d_a[:, None] * h[:, 128:192]
    dinv_ref[...] = jnp.concatenate(
        [d_st[None], d_ts[None], d_a[None], jnp.zeros((5, d_st.shape[0]), _f32)], axis=0)


def _tc_prep1(hist, xp, wcat):
    return pl.pallas_call(
        _prep1_body,
        grid=(GRID,),
        in_specs=[
            pl.BlockSpec((2, 2, BN), lambda i: (0, 0, i)),
            pl.BlockSpec((BN, D), lambda i: (i, 0)),
            pl.BlockSpec((D, 192), lambda i: (0, 0)),
        ],
        out_specs=[
            pl.BlockSpec((BN, 64), lambda i: (i, 0)),
            pl.BlockSpec((BN, 64), lambda i: (i, 0)),
            pl.BlockSpec((BN, 64), lambda i: (i, 0)),
            pl.BlockSpec((8, BN), lambda i: (0, i)),
        ],
        out_shape=[
            jax.ShapeDtypeStruct((NP, 64), _f32),
            jax.ShapeDtypeStruct((NP, 64), _f32),
            jax.ShapeDtypeStruct((NP, 64), _f32),
            jax.ShapeDtypeStruct((8, NP), _f32),
        ],
    )(hist, xp, wcat)


def _mid1_body(pst, pts, pa, tst, tts, ta, dinv, bst, bts, ba, w2, t2o):
    dv = dinv[...]
    h_st = jnp.maximum(dv[0][:, None] * (pst[0] + pst[1] + tst[...]) + bst[...], 0.0)
    h_ts = jnp.maximum(dv[1][:, None] * (pts[0] + pts[1] + tts[...]) + bts[...], 0.0)
    h_a = jnp.maximum(dv[2][:, None] * (pa[0] + pa[1] + ta[...]) + ba[...], 0.0)
    h1 = jnp.concatenate([h_st, h_ts, h_a], axis=1)
    t2o[...] = dv[2][:, None] * jnp.dot(h1, w2[...], preferred_element_type=_f32)


def _tc_mid1(pst, pts, pa, tst, tts, ta, dinv, bst, bts, ba, w2):
    p = pl.BlockSpec((2, BN, 64), lambda i: (0, i, 0))
    t = pl.BlockSpec((BN, 64), lambda i: (i, 0))
    b = pl.BlockSpec((1, 64), lambda i: (0, 0))
    return pl.pallas_call(
        _mid1_body,
        grid=(GRID,),
        in_specs=[p, p, p, t, t, t,
                  pl.BlockSpec((8, BN), lambda i: (0, i)),
                  b, b, b,
                  pl.BlockSpec((192, 128), lambda i: (0, 0))],
        out_specs=pl.BlockSpec((BN, 128), lambda i: (i, 0)),
        out_shape=jax.ShapeDtypeStruct((NP, 128), _f32),
    )(pst, pts, pa, tst, tts, ta, dinv, bst, bts, ba, w2)


def _mid2_body(p2, t2, dinv, b2, w3, t3o):
    da = dinv[...][2]
    h2 = da[:, None] * (p2[0] + p2[1] + t2[...]) + b2[...]
    t3o[...] = da[:, None] * jnp.dot(h2, w3[...], preferred_element_type=_f32)


def _tc_mid2(p2, t2, dinv, b2, w3p):
    return pl.pallas_call(
        _mid2_body,
        grid=(GRID,),
        in_specs=[
            pl.BlockSpec((2, BN, 128), lambda i: (0, i, 0)),
            pl.BlockSpec((BN, 128), lambda i: (i, 0)),
            pl.BlockSpec((8, BN), lambda i: (0, i)),
            pl.BlockSpec((1, 128), lambda i: (0, 0)),
            pl.BlockSpec((128, 64), lambda i: (0, 0)),
        ],
        out_specs=pl.BlockSpec((BN, 64), lambda i: (i, 0)),
        out_shape=jax.ShapeDtypeStruct((NP, 64), _f32),
    )(p2, t2, dinv, b2, w3p)


def _fin_body(p3, t3, dinv, b3, out):
    da = dinv[...][2]
    h3 = da[:, None] * (p3[0] + p3[1] + t3[...]) + b3[...]
    col = lax.broadcasted_iota(jnp.int32, h3.shape, 1)
    valid = col < 40
    hm = jnp.where(valid, h3, jnp.full_like(h3, -jnp.inf))
    mx = jnp.max(hm, axis=1, keepdims=True)
    ex = jnp.where(valid, jnp.exp(h3 - mx), jnp.zeros_like(h3))
    lse = jnp.log(jnp.sum(ex, axis=1, keepdims=True)) + mx
    out[...] = h3 - lse


def _tc_fin(p3, t3, dinv, b3p):
    return pl.pallas_call(
        _fin_body,
        grid=(GRID,),
        in_specs=[
            pl.BlockSpec((2, BN, 64), lambda i: (0, i, 0)),
            pl.BlockSpec((BN, 64), lambda i: (i, 0)),
            pl.BlockSpec((8, BN), lambda i: (0, i)),
            pl.BlockSpec((1, 64), lambda i: (0, 0)),
        ],
        out_specs=pl.BlockSpec((BN, 64), lambda i: (i, 0)),
        out_shape=jax.ShapeDtypeStruct((NP, 64), _f32),
    )(p3, t3, dinv, b3p)


def kernel(x, edge_index, is_reversed, W_st1, b_st1, W_ts1, b_ts1, W1, b1,
           W2, b2, W3, b3):
    src = edge_index[0]
    dst = edge_index[1]
    rev = is_reversed.astype(jnp.int32)
    pad = E_PAD - E
    src_p = jnp.concatenate([src, jnp.zeros((pad,), jnp.int32)]).reshape(NW, NCHUNK, CH)
    dst_p = jnp.concatenate([dst, jnp.full((pad,), TRASH, jnp.int32)]).reshape(NW, NCHUNK, CH)
    rev_p = jnp.concatenate([rev, jnp.zeros((pad,), jnp.int32)]).reshape(NW, NCHUNK, CH)

    hist, ist_p, its_p = _sc_hist(dst_p, rev_p)
    xp = jnp.pad(x, ((0, NP - N), (0, 0)))
    wcat = jnp.concatenate([W_st1, W_ts1, W1], axis=1)
    tst, tts, ta, dinv = _tc_prep1(hist, xp, wcat)
    src_b, dst_b, ist_b, its_b = src_p, dst_p, ist_p, its_p
    pst = _sc_gs64(src_b, ist_b, tst)
    pts = _sc_gs64(src_b, its_b, tts)
    pa = _sc_gs64(src_b, dst_b, ta)
    t2 = _tc_mid1(pst, pts, pa, tst, tts, ta, dinv,
                  b_st1.reshape(1, 64), b_ts1.reshape(1, 64),
                  b1.reshape(1, 64), W2)
    p2 = _sc_gs128(src_b, dst_b, t2)
    w3p = jnp.pad(W3, ((0, 0), (0, 24)))
    t3 = _tc_mid2(p2, t2, dinv, b2.reshape(1, 128), w3p)
    p3 = _sc_gs64(src_b, dst_b, t3)
    b3p = jnp.pad(b3, (0, 24)).reshape(1, 64)
    out = _tc_fin(p3, t3, dinv, b3p)
    return out[:N, :40]


# speedup vs baseline: 1.6808x; 1.1926x over previous
import functools

import jax
import jax.numpy as jnp
from jax import lax
from jax.experimental import pallas as pl
from jax.experimental.pallas import tpu as pltpu
from jax.experimental.pallas import tpu_sc as plsc

N = 10000
E = 320000
D = 128

NC, NS, L = 2, 16, 16
NW = NC * NS
CH = 128
NCHUNK = 79
EPW = NCHUNK * CH
E_PAD = NW * EPW
NP = 10240
TRASH = N
RPT = NP // NS
BN = 512
GRID = NP // BN

_f32 = jnp.float32


def _mesh():
    return plsc.VectorSubcoreMesh(core_axis_name="c", subcore_axis_name="s")


RPT2 = 2 * NP // NS


@functools.partial(
    pl.kernel,
    out_type=(
        jax.ShapeDtypeStruct((NC, 2 * NP), _f32),
        jax.ShapeDtypeStruct((NW, NCHUNK, CH), jnp.int32),
        jax.ShapeDtypeStruct((NW, NCHUNK, CH), jnp.int32),
    ),
    mesh=_mesh(),
    compiler_params=pltpu.CompilerParams(use_tc_tiling_on_sc=False),
    scratch_types=[
        pltpu.VMEM((NCHUNK, CH), jnp.int32),
        pltpu.VMEM((NCHUNK, CH), jnp.int32),
        pltpu.VMEM((NCHUNK, CH), jnp.int32),
        pltpu.VMEM((NCHUNK, CH), jnp.int32),
        pltpu.VMEM((NCHUNK, CH), jnp.int32),
        pltpu.VMEM((CH,), _f32),
        pltpu.VMEM((RPT2,), _f32),
        pltpu.VMEM_SHARED((2 * NP,), _f32),
    ],
)
def _sc_hist(src_hbm, dst_hbm, rev_hbm, out_hbm, gix_hbm, hix_hbm,
             srcb, dstb, revb, gb, hb, ones, zbuf, h_s):
    c = lax.axis_index("c")
    s = lax.axis_index("s")
    w = s * NC + c
    pltpu.sync_copy(src_hbm.at[w], srcb)
    pltpu.sync_copy(dst_hbm.at[w], dstb)
    pltpu.sync_copy(rev_hbm.at[w], revb)
    for i in range(CH // L):
        ones[pl.ds(i * L, L)] = jnp.full((L,), 1.0, _f32)

    def zloop(i, t):
        zbuf[pl.ds(i * L, L)] = jnp.zeros((L,), _f32)
        return t

    lax.fori_loop(0, RPT2 // L, zloop, 0)

    def iloop(j, t):
        for i in range(CH // L):
            sv = srcb[j, pl.ds(i * L, L)]
            d = dstb[j, pl.ds(i * L, L)]
            r = revb[j, pl.ds(i * L, L)]
            off = r * NP
            gb[j, pl.ds(i * L, L)] = sv + off
            hb[j, pl.ds(i * L, L)] = d + off
        return t

    lax.fori_loop(0, NCHUNK, iloop, 0)
    pltpu.sync_copy(gb, gix_hbm.at[w])
    pltpu.sync_copy(hb, hix_hbm.at[w])

    pltpu.sync_copy(zbuf, h_s.at[pl.ds(s * RPT2, RPT2)])
    plsc.subcore_barrier()

    def sloop(j, t):
        pltpu.sync_copy(ones, h_s.at[hb.at[j]], add=True)
        return t

    lax.fori_loop(0, NCHUNK, sloop, 0)
    plsc.subcore_barrier()
    pltpu.sync_copy(h_s.at[pl.ds(s * RPT2, RPT2)], out_hbm.at[c, pl.ds(s * RPT2, RPT2)])


def _make_sc_layer(width, nrows):
    rptk = nrows // NS

    @functools.partial(
        pl.kernel,
        out_type=jax.ShapeDtypeStruct((NC, nrows, width), _f32),
        mesh=_mesh(),
        compiler_params=pltpu.CompilerParams(use_tc_tiling_on_sc=False),
        scratch_types=[
            pltpu.VMEM((NCHUNK, CH), jnp.int32),
            pltpu.VMEM((NCHUNK, CH), jnp.int32),
            pltpu.VMEM((CH, width), _f32),
            pltpu.VMEM((32, width), _f32),
            pltpu.VMEM_SHARED((nrows, width), _f32),
            pltpu.SemaphoreType.DMA,
        ],
    )
    def _sc_layer(src_hbm, dst_hbm, t_hbm, p_hbm,
                  srcb, dstb, rows0, zb, acc, gs0):
        c = lax.axis_index("c")
        s = lax.axis_index("s")
        w = s * NC + c
        pltpu.sync_copy(src_hbm.at[w], srcb)
        pltpu.sync_copy(dst_hbm.at[w], dstb)

        def zbloop(i, t):
            for q in range(width // L):
                zb[i, pl.ds(q * L, L)] = jnp.zeros((L,), _f32)
            return t

        lax.fori_loop(0, 32, zbloop, 0)

        def zsloop(i, t):
            pltpu.sync_copy(zb, acc.at[pl.ds(s * rptk + i * 32, 32)])
            return t

        lax.fori_loop(0, rptk // 32, zsloop, 0)
        plsc.subcore_barrier()

        def mloop(j, t):
            pltpu.async_copy(t_hbm.at[srcb.at[j]], rows0, gs0).wait()
            pltpu.sync_copy(rows0, acc.at[dstb.at[j]], add=True)
            return t

        lax.fori_loop(0, NCHUNK, mloop, 0)
        plsc.subcore_barrier()
        sl = pl.ds(s * rptk, rptk)
        pltpu.sync_copy(acc.at[sl], p_hbm.at[c, sl])

    return _sc_layer


_sc_gs64 = _make_sc_layer(64, NP)
_sc_gsbr = _make_sc_layer(64, 2 * NP)


def _prep1_body(hist_ref, x_ref, wcat_ref, tst_ref, tts_ref, ta_ref, dinv_ref):
    hp = hist_ref[...]
    cnt_fwd = hp[0, 0] + hp[1, 0]
    cnt_rev = hp[0, 1] + hp[1, 1]
    d_st = lax.rsqrt(cnt_fwd + 1.0)
    d_ts = lax.rsqrt(cnt_rev + 1.0)
    d_a = lax.rsqrt(cnt_fwd + cnt_rev + 1.0)
    h = jnp.dot(x_ref[...], wcat_ref[...], preferred_element_type=_f32)
    tst_ref[...] = d_st[:, None] * h[:, :64]
    tts_ref[...] = d_ts[:, None] * h[:, 64:128]
    ta_ref[...] = d_a[:, None] * h[:, 128:192]
    dinv_ref[...] = jnp.concatenate(
        [d_st[None], d_ts[None], d_a[None], jnp.zeros((5, d_st.shape[0]), _f32)], axis=0)


def _tc_prep1(hist, xp, wcat):
    return pl.pallas_call(
        _prep1_body,
        grid=(GRID,),
        in_specs=[
            pl.BlockSpec((2, 2, BN), lambda i: (0, 0, i)),
            pl.BlockSpec((BN, D), lambda i: (i, 0)),
            pl.BlockSpec((D, 192), lambda i: (0, 0)),
        ],
        out_specs=[
            pl.BlockSpec((BN, 64), lambda i: (i, 0)),
            pl.BlockSpec((BN, 64), lambda i: (i, 0)),
            pl.BlockSpec((BN, 64), lambda i: (i, 0)),
            pl.BlockSpec((8, BN), lambda i: (0, i)),
        ],
        out_shape=[
            jax.ShapeDtypeStruct((NP, 64), _f32),
            jax.ShapeDtypeStruct((NP, 64), _f32),
            jax.ShapeDtypeStruct((NP, 64), _f32),
            jax.ShapeDtypeStruct((8, NP), _f32),
        ],
    )(hist, xp, wcat)


def _mid1_body(pst, pts, pa, tst, tts, ta, dinv, bst, bts, ba, w2, *t2o):
    dv = dinv[...]
    h_st = jnp.maximum(dv[0][:, None] * (pst[0] + pst[1] + tst[...]) + bst[...], 0.0)
    h_ts = jnp.maximum(dv[1][:, None] * (pts[0] + pts[1] + tts[...]) + bts[...], 0.0)
    h_a = jnp.maximum(dv[2][:, None] * (pa[0] + pa[1] + ta[...]) + ba[...], 0.0)
    h1 = jnp.concatenate([h_st, h_ts, h_a], axis=1)
    t2 = dv[2][:, None] * jnp.dot(h1, w2[...], preferred_element_type=_f32)
    t2o[0][...] = t2[:, :64]
    t2o[1][...] = t2[:, 64:]


def _tc_mid1(pst, pts, pa, tst, tts, ta, dinv, bst, bts, ba, w2):
    p = pl.BlockSpec((2, BN, 64), lambda i: (0, i, 0))
    pr = pl.BlockSpec((2, BN, 64), lambda i: (0, i + NP // BN, 0))
    t = pl.BlockSpec((BN, 64), lambda i: (i, 0))
    b = pl.BlockSpec((1, 64), lambda i: (0, 0))
    return pl.pallas_call(
        _mid1_body,
        grid=(GRID,),
        in_specs=[p, pr, p, t, t, t,
                  pl.BlockSpec((8, BN), lambda i: (0, i)),
                  b, b, b,
                  pl.BlockSpec((192, 128), lambda i: (0, 0))],
        out_specs=[t, t],
        out_shape=[jax.ShapeDtypeStruct((NP, 64), _f32),
                   jax.ShapeDtypeStruct((NP, 64), _f32)],
    )(pst, pts, pa, tst, tts, ta, dinv, bst, bts, ba, w2)


def _mid2_body(p2a, p2b, t2a, t2b, dinv, b2, w3, t3o):
    da = dinv[...][2]
    h2a = da[:, None] * (p2a[0] + p2a[1] + t2a[...]) + b2[...][:, :64]
    h2b = da[:, None] * (p2b[0] + p2b[1] + t2b[...]) + b2[...][:, 64:]
    h2 = jnp.concatenate([h2a, h2b], axis=1)
    t3o[...] = da[:, None] * jnp.dot(h2, w3[...], preferred_element_type=_f32)


def _tc_mid2(p2a, p2b, t2a, t2b, dinv, b2, w3p):
    p = pl.BlockSpec((2, BN, 64), lambda i: (0, i, 0))
    t = pl.BlockSpec((BN, 64), lambda i: (i, 0))
    return pl.pallas_call(
        _mid2_body,
        grid=(GRID,),
        in_specs=[
            p, p, t, t,
            pl.BlockSpec((8, BN), lambda i: (0, i)),
            pl.BlockSpec((1, 128), lambda i: (0, 0)),
            pl.BlockSpec((128, 64), lambda i: (0, 0)),
        ],
        out_specs=pl.BlockSpec((BN, 64), lambda i: (i, 0)),
        out_shape=jax.ShapeDtypeStruct((NP, 64), _f32),
    )(p2a, p2b, t2a, t2b, dinv, b2, w3p)


def _fin_body(p3, t3, dinv, b3, out):
    da = dinv[...][2]
    h3 = da[:, None] * (p3[0] + p3[1] + t3[...]) + b3[...]
    col = lax.broadcasted_iota(jnp.int32, h3.shape, 1)
    valid = col < 40
    hm = jnp.where(valid, h3, jnp.full_like(h3, -jnp.inf))
    mx = jnp.max(hm, axis=1, keepdims=True)
    ex = jnp.where(valid, jnp.exp(h3 - mx), jnp.zeros_like(h3))
    lse = jnp.log(jnp.sum(ex, axis=1, keepdims=True)) + mx
    out[...] = h3 - lse


def _tc_fin(p3, t3, dinv, b3p):
    return pl.pallas_call(
        _fin_body,
        grid=(GRID,),
        in_specs=[
            pl.BlockSpec((2, BN, 64), lambda i: (0, i, 0)),
            pl.BlockSpec((BN, 64), lambda i: (i, 0)),
            pl.BlockSpec((8, BN), lambda i: (0, i)),
            pl.BlockSpec((1, 64), lambda i: (0, 0)),
        ],
        out_specs=pl.BlockSpec((BN, 64), lambda i: (i, 0)),
        out_shape=jax.ShapeDtypeStruct((NP, 64), _f32),
    )(p3, t3, dinv, b3p)


def kernel(x, edge_index, is_reversed, W_st1, b_st1, W_ts1, b_ts1, W1, b1,
           W2, b2, W3, b3):
    src = edge_index[0]
    dst = edge_index[1]
    rev = is_reversed.astype(jnp.int32)
    pad = E_PAD - E
    src_p = jnp.concatenate([src, jnp.zeros((pad,), jnp.int32)]).reshape(NW, NCHUNK, CH)
    dst_p = jnp.concatenate([dst, jnp.full((pad,), TRASH, jnp.int32)]).reshape(NW, NCHUNK, CH)
    rev_p = jnp.concatenate([rev, jnp.zeros((pad,), jnp.int32)]).reshape(NW, NCHUNK, CH)

    hist2, gix_p, hix_p = _sc_hist(src_p, dst_p, rev_p)
    hist = hist2.reshape(NC, 2, NP)
    xp = jnp.pad(x, ((0, NP - N), (0, 0)))
    wcat = jnp.concatenate([W_st1, W_ts1, W1], axis=1)
    tst, tts, ta, dinv = _tc_prep1(hist, xp, wcat)
    tbr = jnp.concatenate([tst, tts], axis=0)
    pbr = _sc_gsbr(gix_p, hix_p, tbr)
    pa = _sc_gs64(src_p, dst_p, ta)
    t2a, t2b = _tc_mid1(pbr, pbr, pa, tst, tts, ta, dinv,
                        b_st1.reshape(1, 64), b_ts1.reshape(1, 64),
                        b1.reshape(1, 64), W2)
    p2a = _sc_gs64(src_p, dst_p, t2a)
    p2b = _sc_gs64(src_p, dst_p, t2b)
    w3p = jnp.pad(W3, ((0, 0), (0, 24)))
    t3 = _tc_mid2(p2a, p2b, t2a, t2b, dinv, b2.reshape(1, 128), w3p)
    p3 = _sc_gs64(src_p, dst_p, t3)
    b3p = jnp.pad(b3, (0, 24)).reshape(1, 64)
    out = _tc_fin(p3, t3, dinv, b3p)
    return out[:N, :40]
